# Initial kernel scaffold; baseline (speedup 1.0000x reference)
import jax
import jax.numpy as jnp
from jax.experimental import pallas as pl


def kernel(xyz, features, W_ft, b_ft, ln_ft_g, ln_ft_b, conv1_w, conv1_b, bn_g, bn_b, conv2_w, conv2_b, W_fu, b_fu, ln_fu_g, ln_fu_b, alpha, beta):
    raise NotImplementedError("write your pallas kernel here")


# speedup vs baseline: 12.2875x; 12.2875x over previous
import functools

import jax
import jax.numpy as jnp
from jax import lax
from jax.experimental import pallas as pl
from jax.experimental.pallas import tpu as pltpu
from jax.experimental.pallas import tpu_sc as plsc

B, N, K = 8, 2048, 16
OUT_DIM = 128
HID = 64

NC, NS = 2, 16
NW = NC * NS
P_TOTAL = B * N * K
ROWS_PER_W = P_TOTAL // NW
CHUNK = 128
NCHUNK = ROWS_PER_W // CHUNK
TBL_D = 256


def _silu(x):
    return x * (1.0 / (1.0 + jnp.exp(-x)))


_RA = 256


def _knn_body(xr_ref, xa_ref, idx_ref):
    b = pl.program_id(0)
    xr = xr_ref[0]
    xa = xa_ref[0]
    sqr = jnp.sum(xr * xr, axis=1, keepdims=True)
    sqa = jnp.sum(xa * xa, axis=1, keepdims=True)
    d = -2.0 * lax.dot_general(xr, xa, (((1,), (1,)), ((), ())),
                               preferred_element_type=jnp.float32)
    d = d + sqr + sqa.T
    iota = lax.broadcasted_iota(jnp.int32, (_RA, N), 1)
    big = jnp.int32(N)
    for k in range(K):
        m = jnp.min(d, axis=1, keepdims=True)
        am = jnp.min(jnp.where(d == m, iota, big), axis=1)
        idx_ref[0, :, k] = am + b * N
        d = jnp.where(iota == am[:, None], jnp.float32(jnp.inf), d)


def _knn(xyz):
    return pl.pallas_call(
        _knn_body,
        grid=(B, N // _RA),
        in_specs=[
            pl.BlockSpec((1, _RA, 3), lambda b, i: (b, i, 0)),
            pl.BlockSpec((1, N, 3), lambda b, i: (b, 0, 0)),
        ],
        out_specs=pl.BlockSpec((1, _RA, K), lambda b, i: (b, i, 0)),
        out_shape=jax.ShapeDtypeStruct((B, N, K), jnp.int32),
    )(xyz, xyz)


_RB = 1024


def _table_body(f_ref, x_ref, wft_ref, bft_ref, g_ref, b_ref, wl_ref,
                w1_ref, out_ref):
    f = f_ref[...]
    t = lax.dot_general(f, wft_ref[...], (((1,), (1,)), ((), ())),
                        preferred_element_type=jnp.float32) + bft_ref[...]
    m = jnp.mean(t, axis=1, keepdims=True)
    v = jnp.mean((t - m) * (t - m), axis=1, keepdims=True)
    t = (t - m) * lax.rsqrt(v + 1e-5) * g_ref[...] + b_ref[...]
    t = _silu(t)
    g1 = lax.dot_general(t, wl_ref[...], (((1,), (1,)), ((), ())),
                         preferred_element_type=jnp.float32)
    wx = lax.dot_general(x_ref[...], w1_ref[...], (((1,), (1,)), ((), ())),
                         preferred_element_type=jnp.float32)
    pad = jnp.zeros((_RB, TBL_D - OUT_DIM - HID), jnp.float32)
    out_ref[...] = jnp.concatenate([g1, wx, pad], axis=1)


def _table(feat2, xyz2, W_ft, b_ft, ln_g, ln_b, Wl, conv1_w):
    return pl.pallas_call(
        _table_body,
        grid=(B * N // _RB,),
        in_specs=[
            pl.BlockSpec((_RB, 128), lambda i: (i, 0)),
            pl.BlockSpec((_RB, 3), lambda i: (i, 0)),
            pl.BlockSpec((128, 128), lambda i: (0, 0)),
            pl.BlockSpec((1, 128), lambda i: (0, 0)),
            pl.BlockSpec((1, 128), lambda i: (0, 0)),
            pl.BlockSpec((1, 128), lambda i: (0, 0)),
            pl.BlockSpec((128, 128), lambda i: (0, 0)),
            pl.BlockSpec((HID, 3), lambda i: (0, 0)),
        ],
        out_specs=pl.BlockSpec((_RB, TBL_D), lambda i: (i, 0)),
        out_shape=jax.ShapeDtypeStruct((B * N, TBL_D), jnp.float32),
    )(feat2, xyz2, W_ft, b_ft, ln_g, ln_b, Wl, conv1_w)


def _sc_gather(table, idxm):
    mesh = plsc.VectorSubcoreMesh(core_axis_name="c", subcore_axis_name="s")

    @functools.partial(
        pl.kernel, mesh=mesh,
        out_type=jax.ShapeDtypeStruct((P_TOTAL, TBL_D), jnp.float32),
        scratch_types=[
            pltpu.VMEM((NCHUNK, CHUNK), jnp.int32),
            pltpu.VMEM((CHUNK, TBL_D), jnp.float32),
            pltpu.SemaphoreType.DMA,
        ],
    )
    def k(table_hbm, idx_hbm, out_hbm, idx_v, rows_v, sem):
        wid = lax.axis_index("s") * NC + lax.axis_index("c")
        pltpu.sync_copy(idx_hbm.at[wid], idx_v)
        base = wid * ROWS_PER_W

        def body(c, carry):
            pltpu.async_copy(table_hbm.at[idx_v.at[c]], rows_v, sem).wait()
            pltpu.sync_copy(rows_v, out_hbm.at[pl.ds(base + c * CHUNK, CHUNK)])
            return carry

        lax.fori_loop(0, NCHUNK, body, 0)

    return k(table, idxm)


_RC = 4096
_RCP = _RC // K


def _mom_body(y_ref, t_ref, out_ref):
    g3 = y_ref[...].reshape(_RCP, K, TBL_D)
    wxc = t_ref[...][:, OUT_DIM:OUT_DIM + HID]
    h = g3[:, :, OUT_DIM:OUT_DIM + HID] - wxc[:, None, :]
    h2 = h.reshape(_RC, HID)
    s1 = jnp.sum(h2, axis=0, keepdims=True)
    s2 = jnp.sum(h2 * h2, axis=0, keepdims=True)

    @pl.when(pl.program_id(0) == 0)
    def _():
        out_ref[...] = jnp.zeros((2, HID), jnp.float32)

    out_ref[0:1, :] += s1
    out_ref[1:2, :] += s2


def _moments(gathered, table):
    return pl.pallas_call(
        _mom_body,
        grid=(P_TOTAL // _RC,),
        in_specs=[
            pl.BlockSpec((_RC, TBL_D), lambda i: (i, 0)),
            pl.BlockSpec((_RCP, TBL_D), lambda i: (i, 0)),
        ],
        out_specs=pl.BlockSpec((2, HID), lambda i: (0, 0)),
        out_shape=jax.ShapeDtypeStruct((2, HID), jnp.float32),
    )(gathered, table)


_PC = 4096
_PTS = _PC // K


def _fuse_body(y_ref, t_ref, st_ref, mm_ref, c2_ref,
               lg_ref, lb_ref, ab_ref, out_ref):
    g3 = y_ref[...].reshape(_PTS, K, TBL_D)
    wxc = t_ref[...][:, OUT_DIM:OUT_DIM + HID]
    h = g3[:, :, OUT_DIM:OUT_DIM + HID] - wxc[:, None, :]
    st = st_ref[...]
    h = h * st[0:1, None, :] + st[1:2, None, :]
    h = _silu(h)
    z = lax.dot_general(h, mm_ref[...], (((2,), (1,)), ((), ())),
                        preferred_element_type=jnp.float32)
    z = z + g3[:, :, :OUT_DIM] + c2_ref[...][None, 0:1, :]
    m = jnp.mean(z, axis=2, keepdims=True)
    v = jnp.mean((z - m) * (z - m), axis=2, keepdims=True)
    lg = lg_ref[...][None, 0:1, :]
    lb = lb_ref[...][None, 0:1, :]
    z = (z - m) * lax.rsqrt(v + 1e-5) * lg + lb
    fused = _silu(z)
    ab = ab_ref[...]
    fused = ab[None, 0:1, :] * fused + ab[None, 1:2, :]
    logits = jnp.sum(fused, axis=2)
    logits = logits - jnp.max(logits, axis=1, keepdims=True)
    e = jnp.exp(logits)
    w = e / jnp.sum(e, axis=1, keepdims=True)
    out_ref[...] = jnp.sum(w[:, :, None] * fused, axis=1)


def _fuse(gathered, table, st, M, c2, lg, lb, ab):
    return pl.pallas_call(
        _fuse_body,
        grid=(P_TOTAL // _PC,),
        in_specs=[
            pl.BlockSpec((_PC, TBL_D), lambda i: (i, 0)),
            pl.BlockSpec((_PTS, TBL_D), lambda i: (i, 0)),
            pl.BlockSpec((2, HID), lambda i: (0, 0)),
            pl.BlockSpec((128, HID), lambda i: (0, 0)),
            pl.BlockSpec((1, 128), lambda i: (0, 0)),
            pl.BlockSpec((1, 128), lambda i: (0, 0)),
            pl.BlockSpec((1, 128), lambda i: (0, 0)),
            pl.BlockSpec((2, 128), lambda i: (0, 0)),
        ],
        out_specs=pl.BlockSpec((_PTS, OUT_DIM), lambda i: (i, 0)),
        out_shape=jax.ShapeDtypeStruct((B * N, OUT_DIM), jnp.float32),
    )(gathered, table, st, M, c2, lg, lb, ab)


def kernel(xyz, features, W_ft, b_ft, ln_ft_g, ln_ft_b, conv1_w, conv1_b,
           bn_g, bn_b, conv2_w, conv2_b, W_fu, b_fu, ln_fu_g, ln_fu_b,
           alpha, beta):
    Wl = W_fu[:, :OUT_DIM]
    Wr = W_fu[:, OUT_DIM:]
    M = Wr @ conv2_w
    c2 = (Wr @ conv2_b + b_fu)[None, :]

    idx = _knn(xyz)
    idxm = idx.reshape(NW, NCHUNK, CHUNK)

    table = _table(features.reshape(B * N, 128), xyz.reshape(B * N, 3),
                   W_ft, b_ft[None, :], ln_ft_g[None, :], ln_ft_b[None, :],
                   Wl, conv1_w)

    gathered = _sc_gather(table, idxm)

    S = _moments(gathered, table)
    cnt = jnp.float32(P_TOTAL)
    mean_r = S[0] / cnt
    var_h = S[1] / cnt - mean_r * mean_r
    mean_h = mean_r + conv1_b
    s = bn_g * lax.rsqrt(var_h + 1e-5)
    t = bn_b + (conv1_b - mean_h) * s
    st = jnp.stack([s, t], axis=0)

    ab = jnp.concatenate([alpha.reshape(1, 128), beta.reshape(1, 128)], axis=0)

    out = _fuse(gathered, table, st, M, c2,
                ln_fu_g[None, :], ln_fu_b[None, :], ab)
    return out.reshape(B, N, OUT_DIM)


# speedup vs baseline: 12.8187x; 1.0432x over previous
import functools

import jax
import jax.numpy as jnp
from jax import lax
from jax.experimental import pallas as pl
from jax.experimental.pallas import tpu as pltpu
from jax.experimental.pallas import tpu_sc as plsc

B, N, K = 8, 2048, 16
OUT_DIM = 128
HID = 64

NC, NS = 2, 16
NW = NC * NS
P_TOTAL = B * N * K
ROWS_PER_W = P_TOTAL // NW
CHUNK = 128
NCHUNK = ROWS_PER_W // CHUNK
TBL_D = 256


def _silu(x):
    return x * (1.0 / (1.0 + jnp.exp(-x)))


_RA = 256


def _knn_body(xr_ref, xa_ref, idx_ref):
    b = pl.program_id(0)
    xr = xr_ref[0]
    xa = xa_ref[0]
    sqr = jnp.sum(xr * xr, axis=1, keepdims=True)
    sqa = jnp.sum(xa * xa, axis=1, keepdims=True)
    d = -2.0 * lax.dot_general(xr, xa, (((1,), (1,)), ((), ())),
                               preferred_element_type=jnp.float32)
    d = d + sqr + sqa.T
    iota = lax.broadcasted_iota(jnp.int32, (_RA, N), 1)
    big = jnp.int32(N)
    for k in range(K):
        m = jnp.min(d, axis=1, keepdims=True)
        am = jnp.min(jnp.where(d == m, iota, big), axis=1)
        idx_ref[0, :, k] = am + b * N
        d = jnp.where(iota == am[:, None], jnp.float32(jnp.inf), d)


def _knn(xyz):
    return pl.pallas_call(
        _knn_body,
        grid=(B, N // _RA),
        in_specs=[
            pl.BlockSpec((1, _RA, 3), lambda b, i: (b, i, 0)),
            pl.BlockSpec((1, N, 3), lambda b, i: (b, 0, 0)),
        ],
        out_specs=pl.BlockSpec((1, _RA, K), lambda b, i: (b, i, 0)),
        out_shape=jax.ShapeDtypeStruct((B, N, K), jnp.int32),
    )(xyz, xyz)


_RB = 1024


def _table_body(f_ref, x_ref, wft_ref, bft_ref, g_ref, b_ref, wl_ref,
                w1_ref, out_ref):
    f = f_ref[...]
    t = lax.dot_general(f, wft_ref[...], (((1,), (1,)), ((), ())),
                        preferred_element_type=jnp.float32) + bft_ref[...]
    m = jnp.mean(t, axis=1, keepdims=True)
    v = jnp.mean((t - m) * (t - m), axis=1, keepdims=True)
    t = (t - m) * lax.rsqrt(v + 1e-5) * g_ref[...] + b_ref[...]
    t = _silu(t)
    g1 = lax.dot_general(t, wl_ref[...], (((1,), (1,)), ((), ())),
                         preferred_element_type=jnp.float32)
    wx = lax.dot_general(x_ref[...], w1_ref[...], (((1,), (1,)), ((), ())),
                         preferred_element_type=jnp.float32)
    pad = jnp.zeros((_RB, TBL_D - OUT_DIM - HID), jnp.float32)
    out_ref[...] = jnp.concatenate([g1, wx, pad], axis=1)


def _table(feat2, xyz2, W_ft, b_ft, ln_g, ln_b, Wl, conv1_w):
    return pl.pallas_call(
        _table_body,
        grid=(B * N // _RB,),
        in_specs=[
            pl.BlockSpec((_RB, 128), lambda i: (i, 0)),
            pl.BlockSpec((_RB, 3), lambda i: (i, 0)),
            pl.BlockSpec((128, 128), lambda i: (0, 0)),
            pl.BlockSpec((1, 128), lambda i: (0, 0)),
            pl.BlockSpec((1, 128), lambda i: (0, 0)),
            pl.BlockSpec((1, 128), lambda i: (0, 0)),
            pl.BlockSpec((128, 128), lambda i: (0, 0)),
            pl.BlockSpec((HID, 3), lambda i: (0, 0)),
        ],
        out_specs=pl.BlockSpec((_RB, TBL_D), lambda i: (i, 0)),
        out_shape=jax.ShapeDtypeStruct((B * N, TBL_D), jnp.float32),
    )(feat2, xyz2, W_ft, b_ft, ln_g, ln_b, Wl, conv1_w)


def _sc_gather(table, idxm):
    mesh = plsc.VectorSubcoreMesh(core_axis_name="c", subcore_axis_name="s")

    @functools.partial(
        pl.kernel, mesh=mesh,
        out_type=jax.ShapeDtypeStruct((P_TOTAL, TBL_D), jnp.float32),
        scratch_types=[
            pltpu.VMEM((NCHUNK, CHUNK), jnp.int32),
            pltpu.VMEM((CHUNK, TBL_D), jnp.float32),
            pltpu.VMEM((CHUNK, TBL_D), jnp.float32),
            pltpu.SemaphoreType.DMA,
            pltpu.SemaphoreType.DMA,
            pltpu.SemaphoreType.DMA,
            pltpu.SemaphoreType.DMA,
        ],
    )
    def k(table_hbm, idx_hbm, out_hbm, idx_v, r0, r1, sg0, sg1, ss0, ss1):
        wid = lax.axis_index("s") * NC + lax.axis_index("c")
        pltpu.sync_copy(idx_hbm.at[wid], idx_v)
        base = wid * ROWS_PER_W

        def body(i, carry):
            c0 = 2 * i
            c1 = c0 + 1
            h0 = pltpu.async_copy(table_hbm.at[idx_v.at[c0]], r0, sg0)
            h1 = pltpu.async_copy(table_hbm.at[idx_v.at[c1]], r1, sg1)
            h0.wait()
            s0 = pltpu.async_copy(r0, out_hbm.at[pl.ds(base + c0 * CHUNK, CHUNK)], ss0)
            h1.wait()
            s1 = pltpu.async_copy(r1, out_hbm.at[pl.ds(base + c1 * CHUNK, CHUNK)], ss1)
            s0.wait()
            s1.wait()
            return carry

        lax.fori_loop(0, NCHUNK // 2, body, 0)

    return k(table, idxm)


_RC = 4096
_RCP = _RC // K


def _mom_body(y_ref, t_ref, out_ref):
    g3 = y_ref[...].reshape(_RCP, K, OUT_DIM)[:, :, :HID]
    h = g3 - t_ref[...][:, None, :HID]
    h2 = h.reshape(_RC, HID)
    s1 = jnp.sum(h2, axis=0, keepdims=True)
    s2 = jnp.sum(h2 * h2, axis=0, keepdims=True)

    @pl.when(pl.program_id(0) == 0)
    def _():
        out_ref[...] = jnp.zeros((2, HID), jnp.float32)

    out_ref[0:1, :] += s1
    out_ref[1:2, :] += s2


def _moments(gathered, table):
    return pl.pallas_call(
        _mom_body,
        grid=(P_TOTAL // _RC,),
        in_specs=[
            pl.BlockSpec((_RC, OUT_DIM), lambda i: (i, 1)),
            pl.BlockSpec((_RCP, OUT_DIM), lambda i: (i, 1)),
        ],
        out_specs=pl.BlockSpec((2, HID), lambda i: (0, 0)),
        out_shape=jax.ShapeDtypeStruct((2, HID), jnp.float32),
    )(gathered, table)


_PC = 4096
_PTS = _PC // K


def _fuse_body(y1_ref, yw_ref, t_ref, st_ref, mm_ref, c2_ref,
               lg_ref, lb_ref, ab_ref, out_ref):
    g1 = y1_ref[...].reshape(_PTS, K, OUT_DIM)
    wxn = yw_ref[...].reshape(_PTS, K, OUT_DIM)[:, :, :HID]
    h = wxn - t_ref[...][:, None, :HID]
    st = st_ref[...]
    h = h * st[0:1, None, :] + st[1:2, None, :]
    h = _silu(h)
    z = lax.dot_general(h, mm_ref[...], (((2,), (1,)), ((), ())),
                        preferred_element_type=jnp.float32)
    z = z + g1 + c2_ref[...][None, 0:1, :]
    m = jnp.mean(z, axis=2, keepdims=True)
    v = jnp.mean((z - m) * (z - m), axis=2, keepdims=True)
    lg = lg_ref[...][None, 0:1, :]
    lb = lb_ref[...][None, 0:1, :]
    z = (z - m) * lax.rsqrt(v + 1e-5) * lg + lb
    fused = _silu(z)
    ab = ab_ref[...]
    fused = ab[None, 0:1, :] * fused + ab[None, 1:2, :]
    logits = jnp.sum(fused, axis=2)
    logits = logits - jnp.max(logits, axis=1, keepdims=True)
    e = jnp.exp(logits)
    w = e / jnp.sum(e, axis=1, keepdims=True)
    out_ref[...] = jnp.sum(w[:, :, None] * fused, axis=1)


def _fuse(gathered, table, st, M, c2, lg, lb, ab):
    return pl.pallas_call(
        _fuse_body,
        grid=(P_TOTAL // _PC,),
        in_specs=[
            pl.BlockSpec((_PC, OUT_DIM), lambda i: (i, 0)),
            pl.BlockSpec((_PC, OUT_DIM), lambda i: (i, 1)),
            pl.BlockSpec((_PTS, OUT_DIM), lambda i: (i, 1)),
            pl.BlockSpec((2, HID), lambda i: (0, 0)),
            pl.BlockSpec((128, HID), lambda i: (0, 0)),
            pl.BlockSpec((1, 128), lambda i: (0, 0)),
            pl.BlockSpec((1, 128), lambda i: (0, 0)),
            pl.BlockSpec((1, 128), lambda i: (0, 0)),
            pl.BlockSpec((2, 128), lambda i: (0, 0)),
        ],
        out_specs=pl.BlockSpec((_PTS, OUT_DIM), lambda i: (i, 0)),
        out_shape=jax.ShapeDtypeStruct((B * N, OUT_DIM), jnp.float32),
    )(gathered, gathered, table, st, M, c2, lg, lb, ab)


def kernel(xyz, features, W_ft, b_ft, ln_ft_g, ln_ft_b, conv1_w, conv1_b,
           bn_g, bn_b, conv2_w, conv2_b, W_fu, b_fu, ln_fu_g, ln_fu_b,
           alpha, beta):
    Wl = W_fu[:, :OUT_DIM]
    Wr = W_fu[:, OUT_DIM:]
    M = Wr @ conv2_w
    c2 = (Wr @ conv2_b + b_fu)[None, :]

    idx = _knn(xyz)
    idxm = idx.reshape(NW, NCHUNK, CHUNK)

    table = _table(features.reshape(B * N, 128), xyz.reshape(B * N, 3),
                   W_ft, b_ft[None, :], ln_ft_g[None, :], ln_ft_b[None, :],
                   Wl, conv1_w)

    gathered = _sc_gather(table, idxm)

    S = _moments(gathered, table)
    cnt = jnp.float32(P_TOTAL)
    mean_r = S[0] / cnt
    var_h = S[1] / cnt - mean_r * mean_r
    mean_h = mean_r + conv1_b
    s = bn_g * lax.rsqrt(var_h + 1e-5)
    t = bn_b + (conv1_b - mean_h) * s
    st = jnp.stack([s, t], axis=0)

    ab = jnp.concatenate([alpha.reshape(1, 128), beta.reshape(1, 128)], axis=0)

    out = _fuse(gathered, table, st, M, c2,
                ln_fu_g[None, :], ln_fu_b[None, :], ab)
    return out.reshape(B, N, OUT_DIM)


# speedup vs baseline: 13.6576x; 1.0654x over previous
import functools

import jax
import jax.numpy as jnp
from jax import lax
from jax.experimental import pallas as pl
from jax.experimental.pallas import tpu as pltpu
from jax.experimental.pallas import tpu_sc as plsc

B, N, K = 8, 2048, 16
OUT_DIM = 128
HID = 64

NC, NS = 2, 16
NW = NC * NS
P_TOTAL = B * N * K
ROWS_PER_W = P_TOTAL // NW
CHUNK = 128
NCHUNK = ROWS_PER_W // CHUNK
TBL_D = 256


def _silu(x):
    return x * (1.0 / (1.0 + jnp.exp(-x)))


_RA = 256


def _knn_body(xr_ref, xa_ref, idx_ref):
    b = pl.program_id(0)
    xr = xr_ref[0]
    xa = xa_ref[0]
    sqr = jnp.sum(xr * xr, axis=1, keepdims=True)
    sqa = jnp.sum(xa * xa, axis=1, keepdims=True)
    d = -2.0 * lax.dot_general(xr, xa, (((1,), (1,)), ((), ())),
                               preferred_element_type=jnp.float32)
    d = d + sqr + sqa.T
    iota = lax.broadcasted_iota(jnp.int32, (_RA, N), 1)
    for k in range(K):
        am = jnp.argmin(d, axis=1).astype(jnp.int32)
        idx_ref[0, :, k] = am + b * N
        d = jnp.where(iota == am[:, None], jnp.float32(jnp.inf), d)


def _knn(xyz):
    return pl.pallas_call(
        _knn_body,
        grid=(B, N // _RA),
        in_specs=[
            pl.BlockSpec((1, _RA, 3), lambda b, i: (b, i, 0)),
            pl.BlockSpec((1, N, 3), lambda b, i: (b, 0, 0)),
        ],
        out_specs=pl.BlockSpec((1, _RA, K), lambda b, i: (b, i, 0)),
        out_shape=jax.ShapeDtypeStruct((B, N, K), jnp.int32),
    )(xyz, xyz)


_RB = 1024


def _table_body(f_ref, x_ref, wft_ref, bft_ref, g_ref, b_ref, wl_ref,
                w1_ref, out_ref):
    f = f_ref[...]
    t = lax.dot_general(f, wft_ref[...], (((1,), (1,)), ((), ())),
                        preferred_element_type=jnp.float32) + bft_ref[...]
    m = jnp.mean(t, axis=1, keepdims=True)
    v = jnp.mean((t - m) * (t - m), axis=1, keepdims=True)
    t = (t - m) * lax.rsqrt(v + 1e-5) * g_ref[...] + b_ref[...]
    t = _silu(t)
    g1 = lax.dot_general(t, wl_ref[...], (((1,), (1,)), ((), ())),
                         preferred_element_type=jnp.float32)
    wx = lax.dot_general(x_ref[...], w1_ref[...], (((1,), (1,)), ((), ())),
                         preferred_element_type=jnp.float32)
    pad = jnp.zeros((_RB, TBL_D - OUT_DIM - HID), jnp.float32)
    out_ref[...] = jnp.concatenate([g1, wx, pad], axis=1)


def _table(feat2, xyz2, W_ft, b_ft, ln_g, ln_b, Wl, conv1_w):
    return pl.pallas_call(
        _table_body,
        grid=(B * N // _RB,),
        in_specs=[
            pl.BlockSpec((_RB, 128), lambda i: (i, 0)),
            pl.BlockSpec((_RB, 3), lambda i: (i, 0)),
            pl.BlockSpec((128, 128), lambda i: (0, 0)),
            pl.BlockSpec((1, 128), lambda i: (0, 0)),
            pl.BlockSpec((1, 128), lambda i: (0, 0)),
            pl.BlockSpec((1, 128), lambda i: (0, 0)),
            pl.BlockSpec((128, 128), lambda i: (0, 0)),
            pl.BlockSpec((HID, 3), lambda i: (0, 0)),
        ],
        out_specs=pl.BlockSpec((_RB, TBL_D), lambda i: (i, 0)),
        out_shape=jax.ShapeDtypeStruct((B * N, TBL_D), jnp.float32),
    )(feat2, xyz2, W_ft, b_ft, ln_g, ln_b, Wl, conv1_w)


def _sc_gather(table, idxm):
    mesh = plsc.VectorSubcoreMesh(core_axis_name="c", subcore_axis_name="s")

    @functools.partial(
        pl.kernel, mesh=mesh,
        out_type=jax.ShapeDtypeStruct((P_TOTAL, TBL_D), jnp.float32),
        scratch_types=[
            pltpu.VMEM((NCHUNK, CHUNK), jnp.int32),
            pltpu.VMEM((CHUNK, TBL_D), jnp.float32),
            pltpu.VMEM((CHUNK, TBL_D), jnp.float32),
            pltpu.SemaphoreType.DMA,
            pltpu.SemaphoreType.DMA,
            pltpu.SemaphoreType.DMA,
            pltpu.SemaphoreType.DMA,
        ],
    )
    def k(table_hbm, idx_hbm, out_hbm, idx_v, r0, r1, sg0, sg1, ss0, ss1):
        wid = lax.axis_index("s") * NC + lax.axis_index("c")
        pltpu.sync_copy(idx_hbm.at[wid], idx_v)
        base = wid * ROWS_PER_W

        def body(i, carry):
            c0 = 2 * i
            c1 = c0 + 1
            h0 = pltpu.async_copy(table_hbm.at[idx_v.at[c0]], r0, sg0)
            h1 = pltpu.async_copy(table_hbm.at[idx_v.at[c1]], r1, sg1)
            h0.wait()
            s0 = pltpu.async_copy(r0, out_hbm.at[pl.ds(base + c0 * CHUNK, CHUNK)], ss0)
            h1.wait()
            s1 = pltpu.async_copy(r1, out_hbm.at[pl.ds(base + c1 * CHUNK, CHUNK)], ss1)
            s0.wait()
            s1.wait()
            return carry

        lax.fori_loop(0, NCHUNK // 2, body, 0)

    return k(table, idxm)


_RC = 4096
_RCP = _RC // K


def _mom_body(y_ref, t_ref, out_ref):
    g3 = y_ref[...].reshape(_RCP, K, OUT_DIM)[:, :, :HID]
    h = g3 - t_ref[...][:, None, :HID]
    h2 = h.reshape(_RC, HID)
    s1 = jnp.sum(h2, axis=0, keepdims=True)
    s2 = jnp.sum(h2 * h2, axis=0, keepdims=True)

    @pl.when(pl.program_id(0) == 0)
    def _():
        out_ref[...] = jnp.zeros((2, HID), jnp.float32)

    out_ref[0:1, :] += s1
    out_ref[1:2, :] += s2


def _moments(gathered, table):
    return pl.pallas_call(
        _mom_body,
        grid=(P_TOTAL // _RC,),
        in_specs=[
            pl.BlockSpec((_RC, OUT_DIM), lambda i: (i, 1)),
            pl.BlockSpec((_RCP, OUT_DIM), lambda i: (i, 1)),
        ],
        out_specs=pl.BlockSpec((2, HID), lambda i: (0, 0)),
        out_shape=jax.ShapeDtypeStruct((2, HID), jnp.float32),
    )(gathered, table)


_PC = 4096
_PTS = _PC // K


def _fuse_body(y1_ref, yw_ref, t_ref, st_ref, mm_ref, c2_ref,
               lg_ref, lb_ref, ab_ref, out_ref):
    g1 = y1_ref[...].reshape(_PTS, K, OUT_DIM)
    wxn = yw_ref[...].reshape(_PTS, K, OUT_DIM)[:, :, :HID]
    h = wxn - t_ref[...][:, None, :HID]
    st = st_ref[...]
    h = h * st[0:1, None, :] + st[1:2, None, :]
    h = _silu(h)
    z = lax.dot_general(h, mm_ref[...], (((2,), (1,)), ((), ())),
                        preferred_element_type=jnp.float32)
    z = z + g1 + c2_ref[...][None, 0:1, :]
    m = jnp.mean(z, axis=2, keepdims=True)
    v = jnp.mean((z - m) * (z - m), axis=2, keepdims=True)
    lg = lg_ref[...][None, 0:1, :]
    lb = lb_ref[...][None, 0:1, :]
    z = (z - m) * lax.rsqrt(v + 1e-5) * lg + lb
    fused = _silu(z)
    ab = ab_ref[...]
    fused = ab[None, 0:1, :] * fused + ab[None, 1:2, :]
    logits = jnp.sum(fused, axis=2)
    logits = logits - jnp.max(logits, axis=1, keepdims=True)
    e = jnp.exp(logits)
    w = e / jnp.sum(e, axis=1, keepdims=True)
    out_ref[...] = jnp.sum(w[:, :, None] * fused, axis=1)


def _fuse(gathered, table, st, M, c2, lg, lb, ab):
    return pl.pallas_call(
        _fuse_body,
        grid=(P_TOTAL // _PC,),
        in_specs=[
            pl.BlockSpec((_PC, OUT_DIM), lambda i: (i, 0)),
            pl.BlockSpec((_PC, OUT_DIM), lambda i: (i, 1)),
            pl.BlockSpec((_PTS, OUT_DIM), lambda i: (i, 1)),
            pl.BlockSpec((2, HID), lambda i: (0, 0)),
            pl.BlockSpec((128, HID), lambda i: (0, 0)),
            pl.BlockSpec((1, 128), lambda i: (0, 0)),
            pl.BlockSpec((1, 128), lambda i: (0, 0)),
            pl.BlockSpec((1, 128), lambda i: (0, 0)),
            pl.BlockSpec((2, 128), lambda i: (0, 0)),
        ],
        out_specs=pl.BlockSpec((_PTS, OUT_DIM), lambda i: (i, 0)),
        out_shape=jax.ShapeDtypeStruct((B * N, OUT_DIM), jnp.float32),
    )(gathered, gathered, table, st, M, c2, lg, lb, ab)


def kernel(xyz, features, W_ft, b_ft, ln_ft_g, ln_ft_b, conv1_w, conv1_b,
           bn_g, bn_b, conv2_w, conv2_b, W_fu, b_fu, ln_fu_g, ln_fu_b,
           alpha, beta):
    Wl = W_fu[:, :OUT_DIM]
    Wr = W_fu[:, OUT_DIM:]
    M = Wr @ conv2_w
    c2 = (Wr @ conv2_b + b_fu)[None, :]

    idx = _knn(xyz)
    idxm = idx.reshape(NW, NCHUNK, CHUNK)

    table = _table(features.reshape(B * N, 128), xyz.reshape(B * N, 3),
                   W_ft, b_ft[None, :], ln_ft_g[None, :], ln_ft_b[None, :],
                   Wl, conv1_w)

    gathered = _sc_gather(table, idxm)

    S = _moments(gathered, table)
    cnt = jnp.float32(P_TOTAL)
    mean_r = S[0] / cnt
    var_h = S[1] / cnt - mean_r * mean_r
    mean_h = mean_r + conv1_b
    s = bn_g * lax.rsqrt(var_h + 1e-5)
    t = bn_b + (conv1_b - mean_h) * s
    st = jnp.stack([s, t], axis=0)

    ab = jnp.concatenate([alpha.reshape(1, 128), beta.reshape(1, 128)], axis=0)

    out = _fuse(gathered, table, st, M, c2,
                ln_fu_g[None, :], ln_fu_b[None, :], ab)
    return out.reshape(B, N, OUT_DIM)


# speedup vs baseline: 14.3495x; 1.0507x over previous
import functools

import jax
import jax.numpy as jnp
from jax import lax
from jax.experimental import pallas as pl
from jax.experimental.pallas import tpu as pltpu
from jax.experimental.pallas import tpu_sc as plsc

B, N, K = 8, 2048, 16
OUT_DIM = 128
HID = 64

NC, NS = 2, 16
NW = NC * NS
P_TOTAL = B * N * K
ROWS_PER_W = P_TOTAL // NW
CHUNK = 128
NCHUNK = ROWS_PER_W // CHUNK
TBL_D = 256


def _silu(x):
    return x * (1.0 / (1.0 + jnp.exp(-x)))


_RA = 512


def _knn_body(xr_ref, xa_ref, idx_ref):
    b = pl.program_id(0)
    xr = xr_ref[0]
    xa = xa_ref[0]
    sqr = jnp.sum(xr * xr, axis=1, keepdims=True)
    sqa = jnp.sum(xa * xa, axis=1, keepdims=True)
    d = -2.0 * lax.dot_general(xr, xa, (((1,), (1,)), ((), ())),
                               preferred_element_type=jnp.float32)
    d = d + sqr + sqa.T
    iota = lax.broadcasted_iota(jnp.int32, (_RA, N), 1)
    for k in range(K):
        am = jnp.argmin(d, axis=1).astype(jnp.int32)
        idx_ref[0, :, k] = am + b * N
        d = jnp.where(iota == am[:, None], jnp.float32(jnp.inf), d)


def _knn(xyz):
    return pl.pallas_call(
        _knn_body,
        grid=(B, N // _RA),
        in_specs=[
            pl.BlockSpec((1, _RA, 3), lambda b, i: (b, i, 0)),
            pl.BlockSpec((1, N, 3), lambda b, i: (b, 0, 0)),
        ],
        out_specs=pl.BlockSpec((1, _RA, K), lambda b, i: (b, i, 0)),
        out_shape=jax.ShapeDtypeStruct((B, N, K), jnp.int32),
    )(xyz, xyz)


_RB = 1024


def _table_body(f_ref, x_ref, wft_ref, bft_ref, g_ref, b_ref, wl_ref,
                w1_ref, out_ref):
    f = f_ref[...]
    t = lax.dot_general(f, wft_ref[...], (((1,), (1,)), ((), ())),
                        preferred_element_type=jnp.float32) + bft_ref[...]
    m = jnp.mean(t, axis=1, keepdims=True)
    v = jnp.mean((t - m) * (t - m), axis=1, keepdims=True)
    t = (t - m) * lax.rsqrt(v + 1e-5) * g_ref[...] + b_ref[...]
    t = _silu(t)
    g1 = lax.dot_general(t, wl_ref[...], (((1,), (1,)), ((), ())),
                         preferred_element_type=jnp.float32)
    wx = lax.dot_general(x_ref[...], w1_ref[...], (((1,), (1,)), ((), ())),
                         preferred_element_type=jnp.float32)
    pad = jnp.zeros((_RB, TBL_D - OUT_DIM - HID), jnp.float32)
    out_ref[...] = jnp.concatenate([g1, wx, pad], axis=1)


def _table(feat2, xyz2, W_ft, b_ft, ln_g, ln_b, Wl, conv1_w):
    return pl.pallas_call(
        _table_body,
        grid=(B * N // _RB,),
        in_specs=[
            pl.BlockSpec((_RB, 128), lambda i: (i, 0)),
            pl.BlockSpec((_RB, 3), lambda i: (i, 0)),
            pl.BlockSpec((128, 128), lambda i: (0, 0)),
            pl.BlockSpec((1, 128), lambda i: (0, 0)),
            pl.BlockSpec((1, 128), lambda i: (0, 0)),
            pl.BlockSpec((1, 128), lambda i: (0, 0)),
            pl.BlockSpec((128, 128), lambda i: (0, 0)),
            pl.BlockSpec((HID, 3), lambda i: (0, 0)),
        ],
        out_specs=pl.BlockSpec((_RB, TBL_D), lambda i: (i, 0)),
        out_shape=jax.ShapeDtypeStruct((B * N, TBL_D), jnp.float32),
    )(feat2, xyz2, W_ft, b_ft, ln_g, ln_b, Wl, conv1_w)


def _sc_gather(table, idxm):
    mesh = plsc.VectorSubcoreMesh(core_axis_name="c", subcore_axis_name="s")

    @functools.partial(
        pl.kernel, mesh=mesh,
        out_type=jax.ShapeDtypeStruct((P_TOTAL, TBL_D), jnp.float32),
        scratch_types=[
            pltpu.VMEM((NCHUNK, CHUNK), jnp.int32),
            pltpu.VMEM((CHUNK, TBL_D), jnp.float32),
            pltpu.VMEM((CHUNK, TBL_D), jnp.float32),
            pltpu.SemaphoreType.DMA,
            pltpu.SemaphoreType.DMA,
            pltpu.SemaphoreType.DMA,
            pltpu.SemaphoreType.DMA,
        ],
    )
    def k(table_hbm, idx_hbm, out_hbm, idx_v, r0, r1, sg0, sg1, ss0, ss1):
        wid = lax.axis_index("s") * NC + lax.axis_index("c")
        pltpu.sync_copy(idx_hbm.at[wid], idx_v)
        base = wid * ROWS_PER_W

        def body(i, carry):
            c0 = 2 * i
            c1 = c0 + 1
            h0 = pltpu.async_copy(table_hbm.at[idx_v.at[c0]], r0, sg0)
            h1 = pltpu.async_copy(table_hbm.at[idx_v.at[c1]], r1, sg1)
            h0.wait()
            s0 = pltpu.async_copy(r0, out_hbm.at[pl.ds(base + c0 * CHUNK, CHUNK)], ss0)
            h1.wait()
            s1 = pltpu.async_copy(r1, out_hbm.at[pl.ds(base + c1 * CHUNK, CHUNK)], ss1)
            s0.wait()
            s1.wait()
            return carry

        lax.fori_loop(0, NCHUNK // 2, body, 0)

    return k(table, idxm)


_RC = 4096
_RCP = _RC // K


def _mom_body(y_ref, t_ref, out_ref):
    g3 = y_ref[...].reshape(_RCP, K, OUT_DIM)[:, :, :HID]
    h = g3 - t_ref[...][:, None, :HID]
    h2 = h.reshape(_RC, HID)
    s1 = jnp.sum(h2, axis=0, keepdims=True)
    s2 = jnp.sum(h2 * h2, axis=0, keepdims=True)

    @pl.when(pl.program_id(0) == 0)
    def _():
        out_ref[...] = jnp.zeros((2, HID), jnp.float32)

    out_ref[0:1, :] += s1
    out_ref[1:2, :] += s2


def _moments(gathered, table):
    return pl.pallas_call(
        _mom_body,
        grid=(P_TOTAL // _RC,),
        in_specs=[
            pl.BlockSpec((_RC, OUT_DIM), lambda i: (i, 1)),
            pl.BlockSpec((_RCP, OUT_DIM), lambda i: (i, 1)),
        ],
        out_specs=pl.BlockSpec((2, HID), lambda i: (0, 0)),
        out_shape=jax.ShapeDtypeStruct((2, HID), jnp.float32),
    )(gathered, table)


_PC = 4096
_PTS = _PC // K


def _fuse_body(y1_ref, yw_ref, t_ref, st_ref, mm_ref, c2_ref,
               lg_ref, lb_ref, ab_ref, out_ref):
    g1 = y1_ref[...].reshape(_PTS, K, OUT_DIM)
    wxn = yw_ref[...].reshape(_PTS, K, OUT_DIM)[:, :, :HID]
    h = wxn - t_ref[...][:, None, :HID]
    st = st_ref[...]
    h = h * st[0:1, None, :] + st[1:2, None, :]
    h = _silu(h)
    z = lax.dot_general(h, mm_ref[...], (((2,), (1,)), ((), ())),
                        preferred_element_type=jnp.float32)
    z = z + g1 + c2_ref[...][None, 0:1, :]
    m = jnp.mean(z, axis=2, keepdims=True)
    v = jnp.mean((z - m) * (z - m), axis=2, keepdims=True)
    lg = lg_ref[...][None, 0:1, :]
    lb = lb_ref[...][None, 0:1, :]
    z = (z - m) * lax.rsqrt(v + 1e-5) * lg + lb
    fused = _silu(z)
    ab = ab_ref[...]
    fused = ab[None, 0:1, :] * fused + ab[None, 1:2, :]
    logits = jnp.sum(fused, axis=2)
    logits = logits - jnp.max(logits, axis=1, keepdims=True)
    e = jnp.exp(logits)
    w = e / jnp.sum(e, axis=1, keepdims=True)
    out_ref[...] = jnp.sum(w[:, :, None] * fused, axis=1)


def _fuse(gathered, table, st, M, c2, lg, lb, ab):
    return pl.pallas_call(
        _fuse_body,
        grid=(P_TOTAL // _PC,),
        in_specs=[
            pl.BlockSpec((_PC, OUT_DIM), lambda i: (i, 0)),
            pl.BlockSpec((_PC, OUT_DIM), lambda i: (i, 1)),
            pl.BlockSpec((_PTS, OUT_DIM), lambda i: (i, 1)),
            pl.BlockSpec((2, HID), lambda i: (0, 0)),
            pl.BlockSpec((128, HID), lambda i: (0, 0)),
            pl.BlockSpec((1, 128), lambda i: (0, 0)),
            pl.BlockSpec((1, 128), lambda i: (0, 0)),
            pl.BlockSpec((1, 128), lambda i: (0, 0)),
            pl.BlockSpec((2, 128), lambda i: (0, 0)),
        ],
        out_specs=pl.BlockSpec((_PTS, OUT_DIM), lambda i: (i, 0)),
        out_shape=jax.ShapeDtypeStruct((B * N, OUT_DIM), jnp.float32),
    )(gathered, gathered, table, st, M, c2, lg, lb, ab)


def kernel(xyz, features, W_ft, b_ft, ln_ft_g, ln_ft_b, conv1_w, conv1_b,
           bn_g, bn_b, conv2_w, conv2_b, W_fu, b_fu, ln_fu_g, ln_fu_b,
           alpha, beta):
    Wl = W_fu[:, :OUT_DIM]
    Wr = W_fu[:, OUT_DIM:]
    M = Wr @ conv2_w
    c2 = (Wr @ conv2_b + b_fu)[None, :]

    idx = _knn(xyz)
    idxm = idx.reshape(NW, NCHUNK, CHUNK)

    table = _table(features.reshape(B * N, 128), xyz.reshape(B * N, 3),
                   W_ft, b_ft[None, :], ln_ft_g[None, :], ln_ft_b[None, :],
                   Wl, conv1_w)

    gathered = _sc_gather(table, idxm)

    S = _moments(gathered, table)
    cnt = jnp.float32(P_TOTAL)
    mean_r = S[0] / cnt
    var_h = S[1] / cnt - mean_r * mean_r
    mean_h = mean_r + conv1_b
    s = bn_g * lax.rsqrt(var_h + 1e-5)
    t = bn_b + (conv1_b - mean_h) * s
    st = jnp.stack([s, t], axis=0)

    ab = jnp.concatenate([alpha.reshape(1, 128), beta.reshape(1, 128)], axis=0)

    out = _fuse(gathered, table, st, M, c2,
                ln_fu_g[None, :], ln_fu_b[None, :], ab)
    return out.reshape(B, N, OUT_DIM)
